# baseline (speedup 1.0000x reference)
import jax
import jax.numpy as jnp
from jax.experimental import pallas as pl
from jax.experimental.pallas import tpu as pltpu

TILE = 1024
CLIP = 60.0


def _fused_kernel(sizes_ref, x_ref, w_enc_ref, b_enc_ref, v_ref, w_att_ref,
                  w_cls_ref, b_cls_ref, out_ref,
                  acc_ref, l_ref, wenc_bf_ref, v_bf_ref, watt_bf_ref,
                  s_vec_ref, e_vec_ref):
    i = pl.program_id(0)
    nsteps = pl.num_programs(0)
    nb = s_vec_ref.shape[1]

    @pl.when(i == 0)
    def _init():
        acc_ref[...] = jnp.zeros_like(acc_ref)
        l_ref[...] = jnp.zeros_like(l_ref)
        wenc_bf_ref[...] = w_enc_ref[...].astype(jnp.bfloat16)
        v_bf_ref[...] = v_ref[...].astype(jnp.bfloat16)
        watt_bf_ref[...] = w_att_ref[...].astype(jnp.bfloat16)
        lane = jax.lax.broadcasted_iota(jnp.int32, (1, nb), 1)
        s_vec = jnp.zeros((1, nb), jnp.int32)
        e_vec = jnp.zeros((1, nb), jnp.int32)
        for j in range(nb):
            s_vec = jnp.where(lane == j, sizes_ref[j], s_vec)
            e_vec = jnp.where(lane == j, sizes_ref[j + 1], e_vec)
        s_vec_ref[...] = s_vec
        e_vec_ref[...] = e_vec

    x = x_ref[...].astype(jnp.bfloat16)
    hf = jnp.maximum(
        jnp.dot(x, wenc_bf_ref[...], preferred_element_type=jnp.float32)
        + b_enc_ref[...], 0.0)
    h = hf.astype(jnp.bfloat16)
    t = jnp.tanh(jnp.dot(h, v_bf_ref[...],
                         preferred_element_type=jnp.float32))
    s = jnp.dot(t.astype(jnp.bfloat16), watt_bf_ref[...],
                preferred_element_type=jnp.float32)

    e = jnp.exp(jnp.clip(s, -CLIP, CLIP))
    idx = i * TILE + jax.lax.broadcasted_iota(jnp.int32, (TILE, 1), 0)
    inbag = (idx >= s_vec_ref[...]) & (idx < e_vec_ref[...])
    p = jnp.where(inbag, e, 0.0)
    l_ref[...] += jnp.sum(p, axis=0, keepdims=True)
    acc_ref[...] += jax.lax.dot_general(
        p.astype(jnp.bfloat16), h, (((0,), (0,)), ((), ())),
        preferred_element_type=jnp.float32)

    @pl.when(i == nsteps - 1)
    def _fin():
        logits = jax.lax.dot_general(
            w_cls_ref[...], acc_ref[...], (((0,), (1,)), ((), ())),
            preferred_element_type=jnp.float32)
        out_ref[...] = jax.nn.sigmoid(logits / l_ref[...] + b_cls_ref[...])


def kernel(x, bag_sizes, W_enc, b_enc, V, w_att, W_cls, b_cls):
    total, d_in = x.shape
    d_h = W_enc.shape[1]
    d_att = V.shape[1]
    nb = bag_sizes.shape[0] - 1
    nc = W_cls.shape[1]
    grid = total // TILE

    out = pl.pallas_call(
        _fused_kernel,
        grid_spec=pltpu.PrefetchScalarGridSpec(
            num_scalar_prefetch=1,
            grid=(grid,),
            in_specs=[
                pl.BlockSpec((TILE, d_in), lambda i, sz: (i, 0)),
                pl.BlockSpec((d_in, d_h), lambda i, sz: (0, 0)),
                pl.BlockSpec((1, d_h), lambda i, sz: (0, 0)),
                pl.BlockSpec((d_h, d_att), lambda i, sz: (0, 0)),
                pl.BlockSpec((d_att, 1), lambda i, sz: (0, 0)),
                pl.BlockSpec((d_h, nc), lambda i, sz: (0, 0)),
                pl.BlockSpec((1, nc), lambda i, sz: (0, 0)),
            ],
            out_specs=pl.BlockSpec((nc, nb), lambda i, sz: (0, 0)),
            scratch_shapes=[
                pltpu.VMEM((nb, d_h), jnp.float32),
                pltpu.VMEM((1, nb), jnp.float32),
                pltpu.VMEM((d_in, d_h), jnp.bfloat16),
                pltpu.VMEM((d_h, d_att), jnp.bfloat16),
                pltpu.VMEM((d_att, 1), jnp.bfloat16),
                pltpu.VMEM((1, nb), jnp.int32),
                pltpu.VMEM((1, nb), jnp.int32),
            ],
        ),
        out_shape=jax.ShapeDtypeStruct((nc, nb), jnp.float32),
        compiler_params=pltpu.CompilerParams(
            dimension_semantics=("arbitrary",)),
    )(bag_sizes.astype(jnp.int32), x, W_enc, b_enc.reshape(1, d_h),
      V, w_att, W_cls, b_cls.reshape(1, nc))
    return out.T


# speedup vs baseline: 1.0849x; 1.0849x over previous
import jax
import jax.numpy as jnp
from jax.experimental import pallas as pl
from jax.experimental.pallas import tpu as pltpu

TILE = 2048
CLIP = 60.0


def _fused_kernel(sizes_ref, x_ref, w_enc_ref, b_enc_ref, v_ref, w_att_ref,
                  w_cls_ref, b_cls_ref, out_ref,
                  acc_ref, l_ref, wenc_bf_ref, v_bf_ref, watt_bf_ref,
                  s_vec_ref, e_vec_ref):
    i = pl.program_id(0)
    nsteps = pl.num_programs(0)
    nb = s_vec_ref.shape[1]

    @pl.when(i == 0)
    def _init():
        acc_ref[...] = jnp.zeros_like(acc_ref)
        l_ref[...] = jnp.zeros_like(l_ref)
        wenc_bf_ref[...] = w_enc_ref[...].astype(jnp.bfloat16)
        v_bf_ref[...] = v_ref[...].astype(jnp.bfloat16)
        watt_bf_ref[...] = w_att_ref[...].astype(jnp.bfloat16)
        lane = jax.lax.broadcasted_iota(jnp.int32, (1, nb), 1)
        s_vec = jnp.zeros((1, nb), jnp.int32)
        e_vec = jnp.zeros((1, nb), jnp.int32)
        for j in range(nb):
            s_vec = jnp.where(lane == j, sizes_ref[j], s_vec)
            e_vec = jnp.where(lane == j, sizes_ref[j + 1], e_vec)
        s_vec_ref[...] = s_vec
        e_vec_ref[...] = e_vec

    x = x_ref[...].astype(jnp.bfloat16)
    hf = jnp.maximum(
        jnp.dot(x, wenc_bf_ref[...], preferred_element_type=jnp.float32)
        + b_enc_ref[...], 0.0)
    h = hf.astype(jnp.bfloat16)
    t = jnp.tanh(jnp.dot(h, v_bf_ref[...],
                         preferred_element_type=jnp.float32))
    s = jnp.dot(t.astype(jnp.bfloat16), watt_bf_ref[...],
                preferred_element_type=jnp.float32)

    e = jnp.exp(jnp.clip(s, -CLIP, CLIP))
    idx = i * TILE + jax.lax.broadcasted_iota(jnp.int32, (TILE, 1), 0)
    inbag = (idx >= s_vec_ref[...]) & (idx < e_vec_ref[...])
    p = jnp.where(inbag, e, 0.0)
    l_ref[...] += jnp.sum(p, axis=0, keepdims=True)
    acc_ref[...] += jax.lax.dot_general(
        p.astype(jnp.bfloat16), h, (((0,), (0,)), ((), ())),
        preferred_element_type=jnp.float32)

    @pl.when(i == nsteps - 1)
    def _fin():
        logits = jax.lax.dot_general(
            w_cls_ref[...], acc_ref[...], (((0,), (1,)), ((), ())),
            preferred_element_type=jnp.float32)
        out_ref[...] = jax.nn.sigmoid(logits / l_ref[...] + b_cls_ref[...])


def kernel(x, bag_sizes, W_enc, b_enc, V, w_att, W_cls, b_cls):
    total, d_in = x.shape
    d_h = W_enc.shape[1]
    d_att = V.shape[1]
    nb = bag_sizes.shape[0] - 1
    nc = W_cls.shape[1]
    grid = total // TILE

    out = pl.pallas_call(
        _fused_kernel,
        grid_spec=pltpu.PrefetchScalarGridSpec(
            num_scalar_prefetch=1,
            grid=(grid,),
            in_specs=[
                pl.BlockSpec((TILE, d_in), lambda i, sz: (i, 0)),
                pl.BlockSpec((d_in, d_h), lambda i, sz: (0, 0)),
                pl.BlockSpec((1, d_h), lambda i, sz: (0, 0)),
                pl.BlockSpec((d_h, d_att), lambda i, sz: (0, 0)),
                pl.BlockSpec((d_att, 1), lambda i, sz: (0, 0)),
                pl.BlockSpec((d_h, nc), lambda i, sz: (0, 0)),
                pl.BlockSpec((1, nc), lambda i, sz: (0, 0)),
            ],
            out_specs=pl.BlockSpec((nc, nb), lambda i, sz: (0, 0)),
            scratch_shapes=[
                pltpu.VMEM((nb, d_h), jnp.float32),
                pltpu.VMEM((1, nb), jnp.float32),
                pltpu.VMEM((d_in, d_h), jnp.bfloat16),
                pltpu.VMEM((d_h, d_att), jnp.bfloat16),
                pltpu.VMEM((d_att, 1), jnp.bfloat16),
                pltpu.VMEM((1, nb), jnp.int32),
                pltpu.VMEM((1, nb), jnp.int32),
            ],
        ),
        out_shape=jax.ShapeDtypeStruct((nc, nb), jnp.float32),
        compiler_params=pltpu.CompilerParams(
            dimension_semantics=("arbitrary",)),
    )(bag_sizes.astype(jnp.int32), x, W_enc, b_enc.reshape(1, d_h),
      V, w_att, W_cls, b_cls.reshape(1, nc))
    return out.T
